# baseline (speedup 1.0000x reference)
import jax
import jax.numpy as jnp
import numpy as np
from jax.experimental import pallas as pl
from jax.experimental.pallas import tpu as pltpu

N = 1024
FEAT = 128
HID = 128
HEADS = 8
CH = HID // HEADS
CA = CH + 1


def _gat2_kernel(x_ref, adjt_ref, W1_ref, As1_ref, Ad1_ref, b1_ref,
                 W2_ref, As2_ref, Ad2_ref, b2_ref, out_ref):
    adjt = adjt_ref[...]
    row = jax.lax.broadcasted_iota(jnp.int32, (N, N), 0)
    col = jax.lax.broadcasted_iota(jnp.int32, (N, N), 1)
    maskf = jnp.logical_or(row == col, adjt != 0.0).astype(jnp.bfloat16)
    ones_col = jnp.ones((N, 1), dtype=jnp.float32)

    def layer(inp, W_ref, As_ref, Ad_ref, b_ref):
        h = jnp.dot(inp, W_ref[...], preferred_element_type=jnp.float32)
        al_d = jnp.dot(h, Ad_ref[...], preferred_element_type=jnp.float32)
        al_s = jnp.dot(h, As_ref[...], preferred_element_type=jnp.float32)
        al_s_t = jax.lax.dot_general(
            As_ref[...], h, (((0,), (1,)), ((), ())),
            preferred_element_type=jnp.float32)
        S = jnp.max(al_s_t, axis=1, keepdims=True)

        rhs1_bf, u2h_list = [], []
        for hd in range(HEADS):
            du = al_s[:, hd:hd + 1] - S[hd, 0]
            h_aug = jnp.concatenate(
                [h[:, hd * CH:(hd + 1) * CH], ones_col], axis=1)
            rhs1_bf.append((jnp.exp(du) * h_aug).astype(jnp.bfloat16))
            u2h_list.append(jnp.exp(0.2 * du) * h_aug)
        u2h_bf = jnp.concatenate(u2h_list, axis=1).astype(jnp.bfloat16)
        shared = jnp.dot(maskf, u2h_bf,
                         preferred_element_type=jnp.float32)

        outs = []
        for hd in range(HEADS):
            s_row = al_s_t[hd:hd + 1, :].astype(jnp.bfloat16)
            d_col = al_d[:, hd:hd + 1]
            t = d_col.astype(jnp.bfloat16) + s_row
            M1 = jnp.where(t >= 0, maskf, 0)
            rhs = jnp.concatenate(
                [rhs1_bf[hd], u2h_bf[:, hd * CA:(hd + 1) * CA]], axis=1)
            mm = jnp.dot(M1, rhs, preferred_element_type=jnp.float32)
            z = S[hd, 0] + d_col
            mhat = jnp.maximum(z, 0.2 * z)
            v1 = jnp.exp(z - mhat)
            v2 = jnp.exp(0.2 * z - mhat)
            agg = (v1 * mm[:, :CA]
                   + v2 * (shared[:, hd * CA:(hd + 1) * CA] - mm[:, CA:]))
            outs.append(agg[:, :CH] / (agg[:, CH:CA] + 1e-16))
        return jnp.concatenate(outs, axis=1) + b_ref[...]

    h1 = layer(x_ref[...], W1_ref, As1_ref, Ad1_ref, b1_ref)
    h1 = jnp.where(h1 > 0.0, h1, jnp.exp(jnp.minimum(h1, 0.0)) - 1.0)
    h2 = layer(h1, W2_ref, As2_ref, Ad2_ref, b2_ref)
    out_ref[...] = jnp.where(h2 > 0.0, h2, jnp.exp(jnp.minimum(h2, 0.0)) - 1.0)


def _head_proj(a):
    H, C = a.shape
    m = jnp.zeros((H * C, H), dtype=a.dtype)
    idx_r = jnp.arange(H * C)
    idx_c = idx_r // C
    return m.at[idx_r, idx_c].set(a.reshape(-1))


@jax.jit
def kernel(x, adj, W1, a_src1, a_dst1, b1, W2, a_src2, a_dst2, b2):
    As1 = _head_proj(a_src1)
    Ad1 = _head_proj(a_dst1)
    As2 = _head_proj(a_src2)
    Ad2 = _head_proj(a_dst2)
    return pl.pallas_call(
        _gat2_kernel,
        out_shape=jax.ShapeDtypeStruct((N, HID), jnp.float32),
    )(x, adj.T, W1, As1, Ad1, b1.reshape(1, HID),
      W2, As2, Ad2, b2.reshape(1, HID))


# speedup vs baseline: 2.0580x; 2.0580x over previous
import jax
import jax.numpy as jnp
import numpy as np
from jax.experimental import pallas as pl
from jax.experimental.pallas import tpu as pltpu

N = 1024
FEAT = 128
HID = 128
HEADS = 8
CH = HID // HEADS


def _gat2_kernel(x_ref, adjt_ref, W1_ref, As1_ref, Ad1_ref, b1_ref,
                 W2_ref, As2_ref, Ad2_ref, b2_ref, out_ref):
    adjt = adjt_ref[...]
    row = jax.lax.broadcasted_iota(jnp.int32, (N, N), 0)
    col = jax.lax.broadcasted_iota(jnp.int32, (N, N), 1)
    maskf = jnp.where(jnp.logical_or(row == col, adjt != 0.0), 1.0, 0.0)

    def layer(inp, W_ref, As_ref, Ad_ref, b_ref):
        h = jnp.dot(inp, W_ref[...], preferred_element_type=jnp.float32)
        al_d = jnp.dot(h, Ad_ref[...], preferred_element_type=jnp.float32)
        al_s_t = jax.lax.dot_general(
            As_ref[...], h, (((0,), (1,)), ((), ())),
            preferred_element_type=jnp.float32)
        S = jnp.max(al_s_t, axis=1, keepdims=True)
        outs = []
        for hd in range(HEADS):
            s_row = al_s_t[hd:hd + 1, :]
            d_col = al_d[:, hd:hd + 1]
            Sh = S[hd:hd + 1, :]
            z = Sh + d_col
            mhat = jnp.maximum(z, 0.2 * z)
            u1 = jnp.exp(s_row - Sh)
            u2 = jnp.exp(0.2 * (s_row - Sh))
            v1 = jnp.exp(z - mhat)
            v2 = jnp.exp(0.2 * z - mhat)
            t = d_col + s_row
            p = jnp.where(t >= 0.0, v1 * u1, v2 * u2) * maskf
            s = jnp.sum(p, axis=1, keepdims=True)
            h_h = h[:, hd * CH:(hd + 1) * CH]
            o = jnp.dot(p, h_h, preferred_element_type=jnp.float32)
            outs.append(o / (s + 1e-16))
        return jnp.concatenate(outs, axis=1) + b_ref[...]

    h1 = layer(x_ref[...], W1_ref, As1_ref, Ad1_ref, b1_ref)
    h1 = jnp.where(h1 > 0.0, h1, jnp.exp(jnp.minimum(h1, 0.0)) - 1.0)
    h2 = layer(h1, W2_ref, As2_ref, Ad2_ref, b2_ref)
    out_ref[...] = jnp.where(h2 > 0.0, h2, jnp.exp(jnp.minimum(h2, 0.0)) - 1.0)


def _head_proj(a):
    H, C = a.shape
    sel = (jnp.arange(H * C)[:, None] // C == jnp.arange(H)[None, :])
    return sel.astype(a.dtype) * a.reshape(H * C, 1)


@jax.jit
def kernel(x, adj, W1, a_src1, a_dst1, b1, W2, a_src2, a_dst2, b2):
    As1 = _head_proj(a_src1)
    Ad1 = _head_proj(a_dst1)
    As2 = _head_proj(a_src2)
    Ad2 = _head_proj(a_dst2)
    return pl.pallas_call(
        _gat2_kernel,
        out_shape=jax.ShapeDtypeStruct((N, HID), jnp.float32),
    )(x, adj.T, W1, As1, Ad1, b1.reshape(1, HID),
      W2, As2, Ad2, b2.reshape(1, HID))


# speedup vs baseline: 2.2741x; 1.1050x over previous
import jax
import jax.numpy as jnp
import numpy as np
from jax.experimental import pallas as pl
from jax.experimental.pallas import tpu as pltpu

N = 1024
FEAT = 128
HID = 128
HEADS = 8
CH = HID // HEADS


def _expand_proj(a):
    tiled = jnp.concatenate([a] * HEADS, axis=1)
    lane = jax.lax.broadcasted_iota(jnp.int32, (HEADS, HID), 1)
    hrow = jax.lax.broadcasted_iota(jnp.int32, (HEADS, HID), 0)
    return jnp.where(lane // CH == hrow, tiled, 0.0)


def _gat2_kernel(x_ref, adj_ref, W1_ref, as1_ref, ad1_ref, b1_ref,
                 W2_ref, as2_ref, ad2_ref, b2_ref, out_ref):
    adj = adj_ref[...]
    row = jax.lax.broadcasted_iota(jnp.int32, (N, N), 0)
    col = jax.lax.broadcasted_iota(jnp.int32, (N, N), 1)
    maskf = jnp.where(jnp.logical_or(row == col, adj != 0.0), 1.0, 0.0)
    ones_col = jnp.ones((N, 1), dtype=jnp.float32)

    def layer(inp, W_ref, as_ref, ad_ref, b_ref):
        h = jnp.dot(inp, W_ref[...], preferred_element_type=jnp.float32)
        Bs = _expand_proj(as_ref[...])
        Bd = _expand_proj(ad_ref[...])
        al_s = jax.lax.dot_general(h, Bs, (((1,), (1,)), ((), ())),
                                   preferred_element_type=jnp.float32)
        al_d_t = jax.lax.dot_general(Bd, h, (((1,), (1,)), ((), ())),
                                     preferred_element_type=jnp.float32)
        S = jnp.max(al_s, axis=0, keepdims=True)
        outs = []
        for hd in range(HEADS):
            s_col = al_s[:, hd:hd + 1]
            d_row = al_d_t[hd:hd + 1, :]
            Sh = S[:, hd:hd + 1]
            z = Sh + d_row
            mhat = jnp.maximum(z, 0.2 * z)
            u1 = jnp.exp(s_col - Sh)
            u2 = jnp.exp(0.2 * (s_col - Sh))
            v1 = jnp.exp(z - mhat)
            v2 = jnp.exp(0.2 * z - mhat)
            t = s_col + d_row
            p = jnp.where(t >= 0.0, u1 * v1, u2 * v2) * maskf
            h_aug = jnp.concatenate(
                [h[:, hd * CH:(hd + 1) * CH], ones_col], axis=1)
            o_aug = jax.lax.dot_general(p, h_aug, (((0,), (0,)), ((), ())),
                                        preferred_element_type=jnp.float32)
            outs.append(o_aug[:, :CH] / (o_aug[:, CH:CH + 1] + 1e-16))
        return jnp.concatenate(outs, axis=1) + b_ref[...]

    h1 = layer(x_ref[...], W1_ref, as1_ref, ad1_ref, b1_ref)
    h1 = jnp.where(h1 > 0.0, h1, jnp.exp(jnp.minimum(h1, 0.0)) - 1.0)
    h2 = layer(h1, W2_ref, as2_ref, ad2_ref, b2_ref)
    out_ref[...] = jnp.where(h2 > 0.0, h2, jnp.exp(jnp.minimum(h2, 0.0)) - 1.0)


@jax.jit
def kernel(x, adj, W1, a_src1, a_dst1, b1, W2, a_src2, a_dst2, b2):
    return pl.pallas_call(
        _gat2_kernel,
        out_shape=jax.ShapeDtypeStruct((N, HID), jnp.float32),
    )(x, adj, W1, a_src1, a_dst1, b1.reshape(1, HID),
      W2, a_src2, a_dst2, b2.reshape(1, HID))
